# Initial kernel scaffold; baseline (speedup 1.0000x reference)
import jax
import jax.numpy as jnp
from jax import lax
from jax.experimental import pallas as pl
from jax.experimental.pallas import tpu as pltpu
from jax.experimental.pallas import tpu_sc as plsc

NC = 2
NS = 16
LN = 16
CH = 80
ZR = 125


def _fill2d(ref, rows, cols, value):
    v = jnp.full((LN,), value, jnp.float32)

    def body(r, carry):
        for cb in range(cols // LN):
            ref[r, pl.ds(cb * LN, LN)] = v
        return carry

    lax.fori_loop(0, rows, body, 0)


def _make_sc_agg(n, e, d, with_deg):
    ept = e // (NC * NS)
    nchunk = ept // CH
    rpt = n // NS

    out_type = [jax.ShapeDtypeStruct((NC, n, d), jnp.float32)]
    if with_deg:
        out_type.append(jax.ShapeDtypeStruct((NC, n, LN), jnp.float32))

    scratch = [
        pltpu.VMEM((nchunk, CH), jnp.int32),
        pltpu.VMEM((nchunk, CH), jnp.int32),
        pltpu.VMEM((CH, d), jnp.float32),
        pltpu.VMEM((ZR, d), jnp.float32),
        pltpu.SemaphoreType.DMA,
        pltpu.VMEM_SHARED((n, d), jnp.float32),
    ]
    if with_deg:
        scratch.append(pltpu.VMEM((CH, LN), jnp.float32))
        scratch.append(pltpu.VMEM((rpt, LN), jnp.float32))
        scratch.append(pltpu.VMEM_SHARED((n, LN), jnp.float32))

    mesh = plsc.VectorSubcoreMesh(
        core_axis_name="c", subcore_axis_name="s",
        num_cores=NC, num_subcores=NS)

    def body(*refs):
        if with_deg:
            (y_hbm, srcs_hbm, dsts_hbm, p_hbm, deg_hbm,
             src_v, dst_v, rows_v, zb_v, sem, acc_sh,
             ones_v, zd_v, deg_sh) = refs
        else:
            (y_hbm, srcs_hbm, dsts_hbm, p_hbm,
             src_v, dst_v, rows_v, zb_v, sem, acc_sh) = refs

        c = lax.axis_index("c")
        s = lax.axis_index("s")
        wid = c * NS + s

        row0 = wid * nchunk
        pltpu.sync_copy(srcs_hbm.at[pl.ds(row0, nchunk)], src_v)
        pltpu.sync_copy(dsts_hbm.at[pl.ds(row0, nchunk)], dst_v)

        _fill2d(zb_v, ZR, d, 0.0)
        r0 = s * rpt
        for j in range(rpt // ZR):
            pltpu.sync_copy(zb_v, acc_sh.at[pl.ds(r0 + j * ZR, ZR)])
        if with_deg:
            _fill2d(ones_v, CH, LN, 1.0)
            _fill2d(zd_v, rpt, LN, 0.0)
            pltpu.sync_copy(zd_v, deg_sh.at[pl.ds(r0, rpt)])
        plsc.subcore_barrier()

        def chunk(j, carry):
            pltpu.async_copy(y_hbm.at[src_v.at[j]], rows_v, sem).wait()
            pltpu.sync_copy(rows_v, acc_sh.at[dst_v.at[j]], add=True)
            if with_deg:
                pltpu.sync_copy(ones_v, deg_sh.at[dst_v.at[j]], add=True)
            return carry

        lax.fori_loop(0, nchunk, chunk, 0)
        plsc.subcore_barrier()

        pltpu.sync_copy(acc_sh.at[pl.ds(r0, rpt)],
                        p_hbm.at[c, pl.ds(r0, rpt)])
        if with_deg:
            pltpu.sync_copy(deg_sh.at[pl.ds(r0, rpt)],
                            deg_hbm.at[c, pl.ds(r0, rpt)])

    return pl.kernel(body, out_type=out_type, mesh=mesh,
                     scratch_types=scratch)


def _mm_a(x, w):
    n, d = x.shape
    h = w.shape[1]
    blk = 1000

    def body(x_ref, w_ref, o_ref):
        o_ref[...] = jnp.dot(x_ref[...], w_ref[...],
                             preferred_element_type=jnp.float32)

    return pl.pallas_call(
        body,
        grid=(n // blk,),
        in_specs=[pl.BlockSpec((blk, d), lambda i: (i, 0)),
                  pl.BlockSpec((d, h), lambda i: (0, 0))],
        out_specs=pl.BlockSpec((blk, h), lambda i: (i, 0)),
        out_shape=jax.ShapeDtypeStruct((n, h), jnp.float32),
    )(x, w)


def _tc_mid(p, deg, x, bl1, wr1, wl2, wr2):
    n, d = x.shape
    h = wr1.shape[1]
    c = wl2.shape[1]
    blk = 1000

    def body(p_ref, deg_ref, x_ref, bl_ref, wr1_ref, wl2_ref, wr2_ref,
             y2_ref, z2_ref):
        dg = deg_ref[0, :, 0:1] + deg_ref[1, :, 0:1]
        inv = 1.0 / jnp.maximum(dg, 1.0)
        agg = (p_ref[0] + p_ref[1]) * inv
        hid = agg + bl_ref[...] + jnp.dot(
            x_ref[...], wr1_ref[...], preferred_element_type=jnp.float32)
        hid = jnp.maximum(hid, 0.0)
        y2_ref[...] = jnp.dot(hid, wl2_ref[...],
                              preferred_element_type=jnp.float32)
        z2_ref[...] = jnp.dot(hid, wr2_ref[...],
                              preferred_element_type=jnp.float32)

    return pl.pallas_call(
        body,
        grid=(n // blk,),
        in_specs=[pl.BlockSpec((NC, blk, h), lambda i: (0, i, 0)),
                  pl.BlockSpec((NC, blk, LN), lambda i: (0, i, 0)),
                  pl.BlockSpec((blk, d), lambda i: (i, 0)),
                  pl.BlockSpec((1, h), lambda i: (0, 0)),
                  pl.BlockSpec((d, h), lambda i: (0, 0)),
                  pl.BlockSpec((h, c), lambda i: (0, 0)),
                  pl.BlockSpec((h, c), lambda i: (0, 0))],
        out_specs=[pl.BlockSpec((blk, c), lambda i: (i, 0)),
                   pl.BlockSpec((blk, c), lambda i: (i, 0))],
        out_shape=[jax.ShapeDtypeStruct((n, c), jnp.float32),
                   jax.ShapeDtypeStruct((n, c), jnp.float32)],
    )(p, deg, x, bl1, wr1, wl2, wr2)


def _tc_out(q, deg, z2, bl2):
    n = z2.shape[0]
    c = z2.shape[1]
    blk = 1000

    def body(q_ref, deg_ref, z2_ref, bl_ref, o_ref):
        dg = deg_ref[0, :, 0:1] + deg_ref[1, :, 0:1]
        inv = 1.0 / jnp.maximum(dg, 1.0)
        o_ref[...] = (q_ref[0] + q_ref[1]) * inv + bl_ref[...] + z2_ref[...]

    return pl.pallas_call(
        body,
        grid=(n // blk,),
        in_specs=[pl.BlockSpec((NC, blk, c), lambda i: (0, i, 0)),
                  pl.BlockSpec((NC, blk, LN), lambda i: (0, i, 0)),
                  pl.BlockSpec((blk, c), lambda i: (i, 0)),
                  pl.BlockSpec((1, c), lambda i: (0, 0))],
        out_specs=pl.BlockSpec((blk, c), lambda i: (i, 0)),
        out_shape=jax.ShapeDtypeStruct((n, c), jnp.float32),
    )(q, deg, z2, bl2)


def kernel(x, edge_index, Wl1, bl1, Wr1, Wl2, bl2, Wr2):
    n = x.shape[0]
    h = Wl1.shape[1]
    c = Wl2.shape[1]
    e = edge_index.shape[1]

    srcs = edge_index[0].reshape(e // CH, CH)
    dsts = edge_index[1].reshape(e // CH, CH)

    y1 = _mm_a(x, Wl1)
    p, deg = _make_sc_agg(n, e, h, True)(y1, srcs, dsts)
    y2, z2 = _tc_mid(p, deg, x, bl1.reshape(1, h), Wr1, Wl2, Wr2)
    q = _make_sc_agg(n, e, c, False)(y2, srcs, dsts)
    if isinstance(q, (list, tuple)):
        q = q[0]
    return _tc_out(q, deg, z2, bl2.reshape(1, c))


# speedup vs baseline: 6.7877x; 6.7877x over previous
import jax
import jax.numpy as jnp
from jax import lax
from jax.experimental import pallas as pl
from jax.experimental.pallas import tpu as pltpu
from jax.experimental.pallas import tpu_sc as plsc

NC = 2
NS = 16
LN = 16
CH = 80
ZR = 208


def _fill2d(ref, rows, cols, value):
    v = jnp.full((LN,), value, jnp.float32)

    def body(r, carry):
        for cb in range(cols // LN):
            ref[r, pl.ds(cb * LN, LN)] = v
        return carry

    lax.fori_loop(0, rows, body, 0)


def _fill1d(ref, n, value):
    v = jnp.full((LN,), value, jnp.float32)

    def body(k, carry):
        ref[pl.ds(k * LN, LN)] = v
        return carry

    lax.fori_loop(0, n // LN, body, 0)


def _make_sc_agg(n, e, d, with_deg):
    ept = e // (NC * NS)
    nchunk = ept // CH
    w0 = (n // NS) // 8 * 8
    wlast = n - w0 * (NS - 1)
    extra = wlast - w0
    dv = (wlast + LN - 1) // LN * LN

    mesh = plsc.VectorSubcoreMesh(
        core_axis_name="c", subcore_axis_name="s",
        num_cores=NC, num_subcores=NS)

    out_type = [jax.ShapeDtypeStruct((NC, n, d), jnp.float32)]
    scratch = [
        pltpu.VMEM((nchunk, CH), jnp.int32),
        pltpu.VMEM((nchunk, CH), jnp.int32),
        pltpu.VMEM((CH, d), jnp.float32),
        pltpu.VMEM((ZR, d), jnp.float32),
        pltpu.SemaphoreType.DMA,
        pltpu.VMEM_SHARED((n, d), jnp.float32),
    ]
    if with_deg:
        out_type.append(jax.ShapeDtypeStruct((NC, n, LN), jnp.float32))
        scratch += [
            pltpu.VMEM((CH,), jnp.float32),
            pltpu.VMEM((dv,), jnp.float32),
            pltpu.VMEM((dv, LN), jnp.float32),
            pltpu.VMEM_SHARED((n,), jnp.float32),
        ]

    def body(*refs):
        if with_deg:
            (y_hbm, srcs_hbm, dsts_hbm, p_hbm, deg_hbm,
             src_v, dst_v, rows_v, zb_v, sem, acc_sh,
             ones_v, deg_v, db_v, deg_sh) = refs
        else:
            (y_hbm, srcs_hbm, dsts_hbm, p_hbm,
             src_v, dst_v, rows_v, zb_v, sem, acc_sh) = refs

        c = lax.axis_index("c")
        s = lax.axis_index("s")
        wid = c * NS + s

        pltpu.sync_copy(srcs_hbm.at[wid], src_v)
        pltpu.sync_copy(dsts_hbm.at[wid], dst_v)

        _fill2d(zb_v, ZR, d, 0.0)
        off = pl.multiple_of(s * w0, 8)
        for t in range(w0 // ZR):
            pltpu.sync_copy(zb_v, acc_sh.at[pl.ds(off + t * ZR, ZR)])
        if with_deg:
            _fill1d(ones_v, CH, 1.0)
            _fill1d(deg_v, dv, 0.0)
            pltpu.sync_copy(deg_v.at[pl.ds(0, w0)], deg_sh.at[pl.ds(off, w0)])

        @pl.when(s == NS - 1)
        def _():
            pltpu.sync_copy(zb_v.at[pl.ds(0, extra)],
                            acc_sh.at[pl.ds(off + w0, extra)])
            if with_deg:
                pltpu.sync_copy(deg_v.at[pl.ds(0, extra)],
                                deg_sh.at[pl.ds(off + w0, extra)])

        plsc.subcore_barrier()

        def chunk(j, carry):
            pltpu.async_copy(y_hbm.at[src_v.at[j]], rows_v, sem).wait()
            pltpu.sync_copy(rows_v, acc_sh.at[dst_v.at[j]], add=True)
            if with_deg:
                pltpu.sync_copy(ones_v, deg_sh.at[dst_v.at[j]], add=True)
            return carry

        lax.fori_loop(0, nchunk, chunk, 0)
        plsc.subcore_barrier()

        for t in range(w0 // ZR):
            pltpu.sync_copy(acc_sh.at[pl.ds(off + t * ZR, ZR)], zb_v)
            pltpu.sync_copy(zb_v, p_hbm.at[c, pl.ds(off + t * ZR, ZR)])

        @pl.when(s == NS - 1)
        def _():
            pltpu.sync_copy(acc_sh.at[pl.ds(off + w0, extra)],
                            zb_v.at[pl.ds(0, extra)])
            pltpu.sync_copy(zb_v.at[pl.ds(0, extra)],
                            p_hbm.at[c, pl.ds(off + w0, extra)])

        if with_deg:
            pltpu.sync_copy(deg_sh.at[pl.ds(off, w0)],
                            deg_v.at[pl.ds(0, w0)])

            @pl.when(s == NS - 1)
            def _():
                pltpu.sync_copy(deg_sh.at[pl.ds(off + w0, extra)],
                                deg_v.at[pl.ds(w0, extra)])

            def grp(g, carry):
                d16 = deg_v[pl.ds(g * LN, LN)]
                for r in range(LN):
                    db_v[g * LN + r, pl.ds(0, LN)] = (
                        jnp.broadcast_to(d16[r], (LN,)))
                return carry

            lax.fori_loop(0, dv // LN, grp, 0)
            pltpu.sync_copy(db_v.at[pl.ds(0, w0)],
                            deg_hbm.at[c, pl.ds(off, w0)])

            @pl.when(s == NS - 1)
            def _():
                pltpu.sync_copy(db_v.at[pl.ds(w0, extra)],
                                deg_hbm.at[c, pl.ds(off + w0, extra)])

    return pl.kernel(
        body, out_type=out_type, mesh=mesh, scratch_types=scratch,
        compiler_params=pltpu.CompilerParams(use_tc_tiling_on_sc=False,
                                            internal_scratch_in_bytes=65536))


def _mm_a(x, w):
    n, d = x.shape
    h = w.shape[1]
    blk = 1000

    def body(x_ref, w_ref, o_ref):
        o_ref[...] = jnp.dot(x_ref[...], w_ref[...],
                             preferred_element_type=jnp.float32)

    return pl.pallas_call(
        body,
        grid=(n // blk,),
        in_specs=[pl.BlockSpec((blk, d), lambda i: (i, 0)),
                  pl.BlockSpec((d, h), lambda i: (0, 0))],
        out_specs=pl.BlockSpec((blk, h), lambda i: (i, 0)),
        out_shape=jax.ShapeDtypeStruct((n, h), jnp.float32),
    )(x, w)


def _tc_mid(pa, pb, deg, x, bl1, wr1, wl2, wr2):
    n, d = x.shape
    h = wr1.shape[1]
    c = wl2.shape[1]
    hh = h // 2
    blk = 1000

    def body(pa_ref, pb_ref, deg_ref, x_ref, bl_ref, wr1_ref, wl2_ref,
             wr2_ref, y2_ref, z2_ref, inv_ref):
        dg = deg_ref[0, :, :1] + deg_ref[1, :, :1]
        inv = 1.0 / jnp.maximum(dg, 1.0)
        inv_ref[...] = inv
        xw = jnp.dot(x_ref[...], wr1_ref[...],
                     preferred_element_type=jnp.float32)
        h_lo = jnp.maximum(
            (pa_ref[0] + pa_ref[1]) * inv + bl_ref[:, :hh] + xw[:, :hh], 0.0)
        h_hi = jnp.maximum(
            (pb_ref[0] + pb_ref[1]) * inv + bl_ref[:, hh:] + xw[:, hh:], 0.0)
        y2_ref[...] = (
            jnp.dot(h_lo, wl2_ref[:hh], preferred_element_type=jnp.float32)
            + jnp.dot(h_hi, wl2_ref[hh:], preferred_element_type=jnp.float32))
        z2_ref[...] = (
            jnp.dot(h_lo, wr2_ref[:hh], preferred_element_type=jnp.float32)
            + jnp.dot(h_hi, wr2_ref[hh:], preferred_element_type=jnp.float32))

    return pl.pallas_call(
        body,
        grid=(n // blk,),
        in_specs=[pl.BlockSpec((NC, blk, hh), lambda i: (0, i, 0)),
                  pl.BlockSpec((NC, blk, hh), lambda i: (0, i, 0)),
                  pl.BlockSpec((NC, blk, LN), lambda i: (0, i, 0)),
                  pl.BlockSpec((blk, d), lambda i: (i, 0)),
                  pl.BlockSpec((1, h), lambda i: (0, 0)),
                  pl.BlockSpec((d, h), lambda i: (0, 0)),
                  pl.BlockSpec((h, c), lambda i: (0, 0)),
                  pl.BlockSpec((h, c), lambda i: (0, 0))],
        out_specs=[pl.BlockSpec((blk, c), lambda i: (i, 0)),
                   pl.BlockSpec((blk, c), lambda i: (i, 0)),
                   pl.BlockSpec((blk, 1), lambda i: (i, 0))],
        out_shape=[jax.ShapeDtypeStruct((n, c), jnp.float32),
                   jax.ShapeDtypeStruct((n, c), jnp.float32),
                   jax.ShapeDtypeStruct((n, 1), jnp.float32)],
    )(pa, pb, deg, x, bl1, wr1, wl2, wr2)


def _tc_out(q, inv, z2, bl2):
    n, c = z2.shape
    blk = 1000

    def body(q_ref, inv_ref, z2_ref, bl_ref, o_ref):
        o_ref[...] = ((q_ref[0] + q_ref[1]) * inv_ref[...]
                      + bl_ref[...] + z2_ref[...])

    return pl.pallas_call(
        body,
        grid=(n // blk,),
        in_specs=[pl.BlockSpec((NC, blk, c), lambda i: (0, i, 0)),
                  pl.BlockSpec((blk, 1), lambda i: (i, 0)),
                  pl.BlockSpec((blk, c), lambda i: (i, 0)),
                  pl.BlockSpec((1, c), lambda i: (0, 0))],
        out_specs=pl.BlockSpec((blk, c), lambda i: (i, 0)),
        out_shape=jax.ShapeDtypeStruct((n, c), jnp.float32),
    )(q, inv, z2, bl2)


def kernel(x, edge_index, Wl1, bl1, Wr1, Wl2, bl2, Wr2):
    n = x.shape[0]
    h = Wl1.shape[1]
    c = Wl2.shape[1]
    e = edge_index.shape[1]

    nw = NC * NS
    srcs = edge_index[0].reshape(nw, e // (nw * CH), CH)
    dsts = edge_index[1].reshape(nw, e // (nw * CH), CH)

    y1 = _mm_a(x, Wl1)
    pa, deg = _make_sc_agg(n, e, h // 2, True)(y1[:, :h // 2], srcs, dsts)
    pb = _make_sc_agg(n, e, h // 2, False)(y1[:, h // 2:], srcs, dsts)
    if isinstance(pb, (list, tuple)):
        pb = pb[0]
    y2, z2, inv = _tc_mid(pa, pb, deg, x, bl1.reshape(1, h), Wr1, Wl2, Wr2)
    q = _make_sc_agg(n, e, c, False)(y2, srcs, dsts)
    if isinstance(q, (list, tuple)):
        q = q[0]
    return _tc_out(q, inv, z2, bl2.reshape(1, c))
